# Initial kernel scaffold; baseline (speedup 1.0000x reference)
import jax
import jax.numpy as jnp
from jax.experimental import pallas as pl


def kernel(node_features, edge_index, W1, a_src1, a_dst1, b1, W2, a_src2, a_dst2, b2):
    raise NotImplementedError("write your pallas kernel here")


# speedup vs baseline: 21.1135x; 21.1135x over previous
import jax
import jax.numpy as jnp
from jax import lax
from jax.experimental import pallas as pl
from jax.experimental.pallas import tpu as pltpu
from jax.experimental.pallas import tpu_sc as plsc

N, E, D, H, O = 10000, 320000, 128, 128, 64
NP = 10240
NC, NS = 2, 16
CH = 128
SPAN = NP // NS

MAXC = 157
MAXE = MAXC * CH
FH = H // NC


def _stage_edges(src_hbm, dst_hbm, src_v, dst_v, base_e, n_full, has_extra):
    pltpu.sync_copy(src_hbm.at[pl.ds(base_e, n_full)],
                    src_v.at[pl.ds(0, n_full)])
    pltpu.sync_copy(dst_hbm.at[pl.ds(base_e, n_full)],
                    dst_v.at[pl.ds(0, n_full)])

    @pl.when(has_extra)
    def _():
        pltpu.sync_copy(src_hbm.at[pl.ds(base_e + n_full, CH)],
                        src_v.at[pl.ds(n_full, CH)])
        pltpu.sync_copy(dst_hbm.at[pl.ds(base_e + n_full, CH)],
                        dst_v.at[pl.ds(n_full, CH)])


def _edge_scalar_group(src_v, dst_v, as_v, ad_v, l):
    s16 = src_v[pl.ds(l, 16)]
    d16 = dst_v[pl.ds(l, 16)]
    e = plsc.load_gather(as_v, [s16]) + plsc.load_gather(ad_v, [d16])
    e = jnp.where(e >= 0, e, 0.2 * e)
    return s16, d16, jnp.exp(e)


def _sc1_body(src_hbm, dst_hbm, as_hbm, ad_hbm, h1_hbm,
              numer_hbm, denom_hbm,
              src_v, dst_v, as_v, ad_v, idxc_v, dstc_v, exc_v, rows_v,
              half_v, zrow_v, zflat_v, acc_sh, den_sh, sem):
    c = lax.axis_index("c")
    s = lax.axis_index("s")
    nch = jnp.where(s < 4, MAXC, MAXC - 1)
    base_e = (jnp.minimum(s, 4) * MAXC
              + jnp.maximum(s - 4, 0) * (MAXC - 1)) * CH

    _stage_edges(src_hbm, dst_hbm, src_v, dst_v, base_e,
                 (MAXC - 1) * CH, s < 4)
    pltpu.sync_copy(as_hbm, as_v)
    pltpu.sync_copy(ad_hbm, ad_v)

    zero16 = jnp.zeros((16,), jnp.float32)
    for i in range(16):
        for f in range(FH // 16):
            zrow_v[i, pl.ds(f * 16, 16)] = zero16
    for i in range(SPAN // 16):
        zflat_v[pl.ds(i * 16, 16)] = zero16

    row_lo = s * SPAN

    def zacc(k, carry):
        pltpu.sync_copy(zrow_v, acc_sh.at[pl.ds(row_lo + 16 * k, 16)])
        return carry

    lax.fori_loop(0, SPAN // 16, zacc, 0)
    pltpu.sync_copy(zflat_v, den_sh.at[pl.ds(row_lo, SPAN)])
    plsc.subcore_barrier()

    cbase = c * FH

    def chunk(t, carry):
        lbase = t * CH
        exs = []
        for j in range(CH // 16):
            l = lbase + j * 16
            s16, d16, ex = _edge_scalar_group(src_v, dst_v, as_v, ad_v, l)
            exs.append(ex)
            exc_v[pl.ds(j * 16, 16)] = ex
            dstc_v[pl.ds(j * 16, 16)] = d16
            idxc_v[pl.ds(j * 16, 16)] = s16

        pltpu.async_copy(h1_hbm.at[idxc_v], rows_v, sem).wait()

        for j in range(CH // 16):
            for i in range(16):
                sp = jnp.broadcast_to(exs[j][i], (16,))
                r = j * 16 + i
                for f in range(FH // 16):
                    half_v[r, pl.ds(f * 16, 16)] = (
                        rows_v[r, pl.ds(cbase + f * 16, 16)] * sp)

        pltpu.sync_copy(half_v, acc_sh.at[dstc_v], add=True)
        pltpu.sync_copy(exc_v, den_sh.at[dstc_v], add=True)
        return carry

    lax.fori_loop(0, nch, chunk, 0)
    plsc.subcore_barrier()

    pltpu.sync_copy(acc_sh.at[pl.ds(row_lo, SPAN)],
                    numer_hbm.at[c].at[pl.ds(row_lo, SPAN)])

    @pl.when(c == 0)
    def _():
        pltpu.sync_copy(den_sh.at[pl.ds(row_lo, SPAN)],
                        denom_hbm.at[pl.ds(row_lo, SPAN)])


_sc1_call = pl.kernel(
    _sc1_body,
    out_type=(jax.ShapeDtypeStruct((NC, NP, FH), jnp.float32),
              jax.ShapeDtypeStruct((NP,), jnp.float32)),
    mesh=plsc.VectorSubcoreMesh(core_axis_name="c", subcore_axis_name="s",
                                num_cores=NC, num_subcores=NS),
    scratch_types=[
        pltpu.VMEM((MAXE,), jnp.int32),
        pltpu.VMEM((MAXE,), jnp.int32),
        pltpu.VMEM((NP,), jnp.float32),
        pltpu.VMEM((NP,), jnp.float32),
        pltpu.VMEM((CH,), jnp.int32),
        pltpu.VMEM((CH,), jnp.int32),
        pltpu.VMEM((CH,), jnp.float32),
        pltpu.VMEM((CH, H), jnp.float32),
        pltpu.VMEM((CH, FH), jnp.float32),
        pltpu.VMEM((16, FH), jnp.float32),
        pltpu.VMEM((SPAN,), jnp.float32),
        pltpu.VMEM_SHARED((NP, FH), jnp.float32),
        pltpu.VMEM_SHARED((NP,), jnp.float32),
        pltpu.SemaphoreType.DMA,
    ],
    compiler_params=pltpu.CompilerParams(needs_layout_passes=False,
                                        use_tc_tiling_on_sc=False),
)


def _sc2_body(src_hbm, dst_hbm, as_hbm, ad_hbm,
              w_hbm,
              src_v, dst_v, as_v, ad_v, ex_v, den_v, dstc_v, srcc_v, alc_v,
              zflat_v, den_sh, w_sh):
    c = lax.axis_index("c")
    s = lax.axis_index("s")
    nch = jnp.where(s < 4, MAXC, MAXC - 1)
    base_e = (jnp.minimum(s, 4) * MAXC
              + jnp.maximum(s - 4, 0) * (MAXC - 1)) * CH

    _stage_edges(src_hbm, dst_hbm, src_v, dst_v, base_e,
                 (MAXC - 1) * CH, s < 4)
    pltpu.sync_copy(as_hbm, as_v)
    pltpu.sync_copy(ad_hbm, ad_v)

    zero16 = jnp.zeros((16,), jnp.float32)
    for i in range(SPAN // 16):
        zflat_v[pl.ds(i * 16, 16)] = zero16

    row_lo = s * SPAN
    pltpu.sync_copy(zflat_v, den_sh.at[pl.ds(row_lo, SPAN)])
    pltpu.sync_copy(zflat_v, w_sh.at[pl.ds(row_lo, SPAN)])
    plsc.subcore_barrier()

    def pass_a(t, carry):
        lbase = t * CH
        for j in range(CH // 16):
            l = lbase + j * 16
            _, d16, ex = _edge_scalar_group(src_v, dst_v, as_v, ad_v, l)
            ex_v[pl.ds(l, 16)] = ex
            dstc_v[pl.ds(j * 16, 16)] = d16
        pltpu.sync_copy(ex_v.at[pl.ds(lbase, CH)], den_sh.at[dstc_v], add=True)
        return carry

    lax.fori_loop(0, nch, pass_a, 0)
    plsc.subcore_barrier()
    pltpu.sync_copy(den_sh, den_v)

    def pass_b(t, carry):
        lbase = t * CH
        for j in range(CH // 16):
            l = lbase + j * 16
            d16 = dst_v[pl.ds(l, 16)]
            dv = plsc.load_gather(den_v, [d16])
            dv = jnp.where(dv > 0, dv, 1.0)
            alc_v[pl.ds(j * 16, 16)] = ex_v[pl.ds(l, 16)] / dv
            srcc_v[pl.ds(j * 16, 16)] = src_v[pl.ds(l, 16)]
        pltpu.sync_copy(alc_v, w_sh.at[srcc_v], add=True)
        return carry

    lax.fori_loop(0, nch, pass_b, 0)
    plsc.subcore_barrier()

    @pl.when(c == 0)
    def _():
        pltpu.sync_copy(w_sh.at[pl.ds(row_lo, SPAN)],
                        w_hbm.at[pl.ds(row_lo, SPAN)])


_sc2_call = pl.kernel(
    _sc2_body,
    out_type=jax.ShapeDtypeStruct((NP,), jnp.float32),
    mesh=plsc.VectorSubcoreMesh(core_axis_name="c", subcore_axis_name="s",
                                num_cores=NC, num_subcores=NS),
    scratch_types=[
        pltpu.VMEM((MAXE,), jnp.int32),
        pltpu.VMEM((MAXE,), jnp.int32),
        pltpu.VMEM((NP,), jnp.float32),
        pltpu.VMEM((NP,), jnp.float32),
        pltpu.VMEM((MAXE,), jnp.float32),
        pltpu.VMEM((NP,), jnp.float32),
        pltpu.VMEM((CH,), jnp.int32),
        pltpu.VMEM((CH,), jnp.int32),
        pltpu.VMEM((CH,), jnp.float32),
        pltpu.VMEM((SPAN,), jnp.float32),
        pltpu.VMEM_SHARED((NP,), jnp.float32),
        pltpu.VMEM_SHARED((NP,), jnp.float32),
    ],
    compiler_params=pltpu.CompilerParams(needs_layout_passes=False,
                                        use_tc_tiling_on_sc=False),
)


def _tc1_body(x_ref, w1_ref, av_ref, h_ref, sd_ref):
    h = jnp.dot(x_ref[...], w1_ref[...],
                preferred_element_type=jnp.float32,
                precision=lax.Precision.HIGHEST)
    h_ref[...] = h
    sd_ref[...] = jnp.dot(h, av_ref[...],
                          preferred_element_type=jnp.float32,
                          precision=lax.Precision.HIGHEST)


_tc1_call = pl.pallas_call(
    _tc1_body,
    out_shape=(jax.ShapeDtypeStruct((N, H), jnp.float32),
               jax.ShapeDtypeStruct((N, 2), jnp.float32)),
)


def _tc2_body(num_ref, den_ref, b1_ref, w2_ref, av_ref, h2_ref, sd_ref):
    den = den_ref[...]
    d = jnp.where(den > 0, den, 1.0)
    x = num_ref[...] / d + b1_ref[...]
    x = jnp.where(x > 0, x, jnp.exp(x) - 1.0)
    h2 = jnp.dot(x, w2_ref[...],
                 preferred_element_type=jnp.float32,
                 precision=lax.Precision.HIGHEST)
    h2_ref[...] = h2
    sd_ref[...] = jnp.dot(h2, av_ref[...],
                          preferred_element_type=jnp.float32,
                          precision=lax.Precision.HIGHEST)


_tc2_call = pl.pallas_call(
    _tc2_body,
    out_shape=(jax.ShapeDtypeStruct((NP, O), jnp.float32),
               jax.ShapeDtypeStruct((NP, 2), jnp.float32)),
)


def _tc3_body(h2_ref, w_ref, b2_ref, out_ref):
    acc = jnp.sum(h2_ref[...] * w_ref[...], axis=0, keepdims=True)
    out_ref[...] = acc * jnp.float32(1.0 / N) + b2_ref[...]


_tc3_call = pl.pallas_call(
    _tc3_body,
    out_shape=jax.ShapeDtypeStruct((1, O), jnp.float32),
)


def kernel(node_features, edge_index, W1, a_src1, a_dst1, b1,
           W2, a_src2, a_dst2, b2):
    src = edge_index[0].astype(jnp.int32)
    dst = edge_index[1].astype(jnp.int32)

    av1 = jnp.stack([a_src1, a_dst1], axis=1)
    h1, sd1 = _tc1_call(node_features, W1, av1)
    as1 = jnp.pad(sd1[:, 0], (0, NP - N))
    ad1 = jnp.pad(sd1[:, 1], (0, NP - N))

    numer, denom = _sc1_call(src, dst, as1, ad1, h1)
    nfull = jnp.concatenate([numer[0], numer[1]], axis=1)

    av2 = jnp.stack([a_src2, a_dst2], axis=1)
    h2, sd2 = _tc2_call(nfull, denom.reshape(NP, 1), b1.reshape(1, H),
                        W2, av2)

    w = _sc2_call(src, dst, sd2[:, 0], sd2[:, 1])

    out = _tc3_call(h2, w.reshape(NP, 1), b2.reshape(1, O))
    return out.reshape(O)


# speedup vs baseline: 51.7008x; 2.4487x over previous
import jax
import jax.numpy as jnp
from jax import lax
from jax.experimental import pallas as pl
from jax.experimental.pallas import tpu as pltpu
from jax.experimental.pallas import tpu_sc as plsc

N, E, D, H, O = 10000, 320000, 128, 128, 64
NP = 10240
NC, NS = 2, 16
CH = 128
SPAN = NP // NS

MAXC = 157
MAXE = MAXC * CH
FH = H // NC


def _stage_edges(src_hbm, dst_hbm, src_v, dst_v, base_e, n_full, has_extra):
    pltpu.sync_copy(src_hbm.at[pl.ds(base_e, n_full)],
                    src_v.at[pl.ds(0, n_full)])
    pltpu.sync_copy(dst_hbm.at[pl.ds(base_e, n_full)],
                    dst_v.at[pl.ds(0, n_full)])

    @pl.when(has_extra)
    def _():
        pltpu.sync_copy(src_hbm.at[pl.ds(base_e + n_full, CH)],
                        src_v.at[pl.ds(n_full, CH)])
        pltpu.sync_copy(dst_hbm.at[pl.ds(base_e + n_full, CH)],
                        dst_v.at[pl.ds(n_full, CH)])


def _edge_scalar_group(src_v, dst_v, as_v, ad_v, l):
    s16 = src_v[pl.ds(l, 16)]
    d16 = dst_v[pl.ds(l, 16)]
    e = plsc.load_gather(as_v, [s16]) + plsc.load_gather(ad_v, [d16])
    e = jnp.where(e >= 0, e, 0.2 * e)
    return s16, d16, jnp.exp(e)


def _sc1_body(src_hbm, dst_hbm, as_hbm, ad_hbm, h1s_hbm,
              numer_hbm, denom_hbm,
              src_v, dst_v, as_v, ad_v,
              idxc0_v, dstc0_v, exc0_v, rows0_v,
              idxc1_v, dstc1_v, exc1_v, rows1_v,
              zrow_v, zflat_v, acc_sh, den_sh,
              gsem0, rsem0, gsem1, rsem1):
    c = lax.axis_index("c")
    s = lax.axis_index("s")
    nch = jnp.where(s < 4, MAXC, MAXC - 1)
    base_e = (jnp.minimum(s, 4) * MAXC
              + jnp.maximum(s - 4, 0) * (MAXC - 1)) * CH

    _stage_edges(src_hbm, dst_hbm, src_v, dst_v, base_e,
                 (MAXC - 1) * CH, s < 4)
    pltpu.sync_copy(as_hbm, as_v)
    pltpu.sync_copy(ad_hbm, ad_v)

    zero16 = jnp.zeros((16,), jnp.float32)
    for i in range(16):
        for f in range(FH // 16):
            zrow_v[i, pl.ds(f * 16, 16)] = zero16
    for i in range(SPAN // 16):
        zflat_v[pl.ds(i * 16, 16)] = zero16

    row_lo = s * SPAN

    def zacc(k, carry):
        pltpu.sync_copy(zrow_v, acc_sh.at[pl.ds(row_lo + 16 * k, 16)])
        return carry

    lax.fori_loop(0, SPAN // 16, zacc, 0)
    pltpu.sync_copy(zflat_v, den_sh.at[pl.ds(row_lo, SPAN)])
    plsc.subcore_barrier()

    coff = c * N

    B = ((idxc0_v, dstc0_v, exc0_v, rows0_v, gsem0, rsem0),
         (idxc1_v, dstc1_v, exc1_v, rows1_v, gsem1, rsem1))

    def build(t, b):
        idxc, dstc, exc = b[0], b[1], b[2]
        lbase = t * CH
        for j in range(CH // 16):
            l = lbase + j * 16
            s16, d16, ex = _edge_scalar_group(src_v, dst_v, as_v, ad_v, l)
            exc[pl.ds(j * 16, 16)] = ex
            dstc[pl.ds(j * 16, 16)] = d16
            idxc[pl.ds(j * 16, 16)] = s16 + coff
        pltpu.async_copy(h1s_hbm.at[idxc], b[3], b[4])

    def drain_scatters(b):
        pltpu.make_async_copy(b[3], acc_sh.at[b[1]], b[5]).wait()

    def process(b):
        idxc, dstc, exc, rows = b[0], b[1], b[2], b[3]
        pltpu.make_async_copy(h1s_hbm.at[idxc], rows, b[4]).wait()
        for j in range(CH // 16):
            ex16 = exc[pl.ds(j * 16, 16)]
            for i in range(16):
                sp = jnp.broadcast_to(ex16[i], (16,))
                r = j * 16 + i
                for f in range(FH // 16):
                    rows[r, pl.ds(f * 16, 16)] = rows[r, pl.ds(f * 16, 16)] * sp
        pltpu.async_copy(rows, acc_sh.at[dstc], b[5], add=True)
        pltpu.sync_copy(exc, den_sh.at[dstc], add=True)

    build(0, B[0])

    def chunk(t, carry):
        for p in range(2):
            @pl.when(t % 2 == p)
            def _():
                @pl.when(t + 1 < nch)
                def _():
                    @pl.when(t >= 1)
                    def _():
                        drain_scatters(B[1 - p])
                    build(t + 1, B[1 - p])
                process(B[p])
        return carry

    lax.fori_loop(0, nch, chunk, 0)
    drain_scatters(B[0])
    drain_scatters(B[1])
    plsc.subcore_barrier()

    pltpu.sync_copy(acc_sh.at[pl.ds(row_lo, SPAN)],
                    numer_hbm.at[c].at[pl.ds(row_lo, SPAN)])

    @pl.when(c == 0)
    def _():
        pltpu.sync_copy(den_sh.at[pl.ds(row_lo, SPAN)],
                        denom_hbm.at[pl.ds(row_lo, SPAN)])


_sc1_call = pl.kernel(
    _sc1_body,
    out_type=(jax.ShapeDtypeStruct((NC, NP, FH), jnp.float32),
              jax.ShapeDtypeStruct((NP,), jnp.float32)),
    mesh=plsc.VectorSubcoreMesh(core_axis_name="c", subcore_axis_name="s",
                                num_cores=NC, num_subcores=NS),
    scratch_types=[
        pltpu.VMEM((MAXE,), jnp.int32),
        pltpu.VMEM((MAXE,), jnp.int32),
        pltpu.VMEM((NP,), jnp.float32),
        pltpu.VMEM((NP,), jnp.float32),
        pltpu.VMEM((CH,), jnp.int32),
        pltpu.VMEM((CH,), jnp.int32),
        pltpu.VMEM((CH,), jnp.float32),
        pltpu.VMEM((CH, FH), jnp.float32),
        pltpu.VMEM((CH,), jnp.int32),
        pltpu.VMEM((CH,), jnp.int32),
        pltpu.VMEM((CH,), jnp.float32),
        pltpu.VMEM((CH, FH), jnp.float32),
        pltpu.VMEM((16, FH), jnp.float32),
        pltpu.VMEM((SPAN,), jnp.float32),
        pltpu.VMEM_SHARED((NP, FH), jnp.float32),
        pltpu.VMEM_SHARED((NP,), jnp.float32),
        pltpu.SemaphoreType.DMA,
        pltpu.SemaphoreType.DMA,
        pltpu.SemaphoreType.DMA,
        pltpu.SemaphoreType.DMA,
    ],
    compiler_params=pltpu.CompilerParams(needs_layout_passes=False,
                                        use_tc_tiling_on_sc=False),
)


def _sc2_body(src_hbm, dst_hbm, as_hbm, ad_hbm,
              w_hbm,
              src_v, dst_v, as_v, ad_v, ex_v, den_v, dstc_v, srcc_v, alc_v,
              zflat_v, den_sh, w_sh):
    c = lax.axis_index("c")
    s = lax.axis_index("s")
    nch = jnp.where(s < 4, MAXC, MAXC - 1)
    base_e = (jnp.minimum(s, 4) * MAXC
              + jnp.maximum(s - 4, 0) * (MAXC - 1)) * CH

    _stage_edges(src_hbm, dst_hbm, src_v, dst_v, base_e,
                 (MAXC - 1) * CH, s < 4)
    pltpu.sync_copy(as_hbm, as_v)
    pltpu.sync_copy(ad_hbm, ad_v)

    zero16 = jnp.zeros((16,), jnp.float32)
    for i in range(SPAN // 16):
        zflat_v[pl.ds(i * 16, 16)] = zero16

    row_lo = s * SPAN
    pltpu.sync_copy(zflat_v, den_sh.at[pl.ds(row_lo, SPAN)])
    pltpu.sync_copy(zflat_v, w_sh.at[pl.ds(row_lo, SPAN)])
    plsc.subcore_barrier()

    def pass_a(t, carry):
        lbase = t * CH
        for j in range(CH // 16):
            l = lbase + j * 16
            _, d16, ex = _edge_scalar_group(src_v, dst_v, as_v, ad_v, l)
            ex_v[pl.ds(l, 16)] = ex
            dstc_v[pl.ds(j * 16, 16)] = d16
        pltpu.sync_copy(ex_v.at[pl.ds(lbase, CH)], den_sh.at[dstc_v], add=True)
        return carry

    lax.fori_loop(0, nch, pass_a, 0)
    plsc.subcore_barrier()
    pltpu.sync_copy(den_sh, den_v)

    def pass_b(t, carry):
        lbase = t * CH
        for j in range(CH // 16):
            l = lbase + j * 16
            d16 = dst_v[pl.ds(l, 16)]
            dv = plsc.load_gather(den_v, [d16])
            dv = jnp.where(dv > 0, dv, 1.0)
            alc_v[pl.ds(j * 16, 16)] = ex_v[pl.ds(l, 16)] / dv
            srcc_v[pl.ds(j * 16, 16)] = src_v[pl.ds(l, 16)]
        pltpu.sync_copy(alc_v, w_sh.at[srcc_v], add=True)
        return carry

    lax.fori_loop(0, nch, pass_b, 0)
    plsc.subcore_barrier()

    @pl.when(c == 0)
    def _():
        pltpu.sync_copy(w_sh.at[pl.ds(row_lo, SPAN)],
                        w_hbm.at[pl.ds(row_lo, SPAN)])


_sc2_call = pl.kernel(
    _sc2_body,
    out_type=jax.ShapeDtypeStruct((NP,), jnp.float32),
    mesh=plsc.VectorSubcoreMesh(core_axis_name="c", subcore_axis_name="s",
                                num_cores=NC, num_subcores=NS),
    scratch_types=[
        pltpu.VMEM((MAXE,), jnp.int32),
        pltpu.VMEM((MAXE,), jnp.int32),
        pltpu.VMEM((NP,), jnp.float32),
        pltpu.VMEM((NP,), jnp.float32),
        pltpu.VMEM((MAXE,), jnp.float32),
        pltpu.VMEM((NP,), jnp.float32),
        pltpu.VMEM((CH,), jnp.int32),
        pltpu.VMEM((CH,), jnp.int32),
        pltpu.VMEM((CH,), jnp.float32),
        pltpu.VMEM((SPAN,), jnp.float32),
        pltpu.VMEM_SHARED((NP,), jnp.float32),
        pltpu.VMEM_SHARED((NP,), jnp.float32),
    ],
    compiler_params=pltpu.CompilerParams(needs_layout_passes=False,
                                        use_tc_tiling_on_sc=False),
)


def _tc1_body(x_ref, w1_ref, av_ref, h_ref, sd_ref):
    h = jnp.dot(x_ref[...], w1_ref[...],
                preferred_element_type=jnp.float32,
                precision=lax.Precision.HIGHEST)
    h_ref[...] = h
    sd_ref[...] = jnp.dot(h, av_ref[...],
                          preferred_element_type=jnp.float32,
                          precision=lax.Precision.HIGHEST)


_tc1_call = pl.pallas_call(
    _tc1_body,
    out_shape=(jax.ShapeDtypeStruct((N, H), jnp.float32),
               jax.ShapeDtypeStruct((N, 2), jnp.float32)),
)


def _tc2_body(num_ref, den_ref, b1_ref, w2_ref, av_ref, h2_ref, sd_ref):
    den = den_ref[...]
    d = jnp.where(den > 0, den, 1.0)
    x = num_ref[...] / d + b1_ref[...]
    x = jnp.where(x > 0, x, jnp.exp(x) - 1.0)
    h2 = jnp.dot(x, w2_ref[...],
                 preferred_element_type=jnp.float32,
                 precision=lax.Precision.HIGHEST)
    h2_ref[...] = h2
    sd_ref[...] = jnp.dot(h2, av_ref[...],
                          preferred_element_type=jnp.float32,
                          precision=lax.Precision.HIGHEST)


_tc2_call = pl.pallas_call(
    _tc2_body,
    out_shape=(jax.ShapeDtypeStruct((NP, O), jnp.float32),
               jax.ShapeDtypeStruct((NP, 2), jnp.float32)),
)


def _tc3_body(h2_ref, w_ref, b2_ref, out_ref):
    acc = jnp.sum(h2_ref[...] * w_ref[...], axis=0, keepdims=True)
    out_ref[...] = acc * jnp.float32(1.0 / N) + b2_ref[...]


_tc3_call = pl.pallas_call(
    _tc3_body,
    out_shape=jax.ShapeDtypeStruct((1, O), jnp.float32),
)


def kernel(node_features, edge_index, W1, a_src1, a_dst1, b1,
           W2, a_src2, a_dst2, b2):
    src = edge_index[0].astype(jnp.int32)
    dst = edge_index[1].astype(jnp.int32)

    av1 = jnp.stack([a_src1, a_dst1], axis=1)
    h1, sd1 = _tc1_call(node_features, W1, av1)
    as1 = jnp.pad(sd1[:, 0], (0, NP - N))
    ad1 = jnp.pad(sd1[:, 1], (0, NP - N))

    h1s = jnp.concatenate([h1[:, :FH], h1[:, FH:]], axis=0)
    numer, denom = _sc1_call(src, dst, as1, ad1, h1s)
    nfull = jnp.concatenate([numer[0], numer[1]], axis=1)

    av2 = jnp.stack([a_src2, a_dst2], axis=1)
    h2, sd2 = _tc2_call(nfull, denom.reshape(NP, 1), b1.reshape(1, H),
                        W2, av2)

    w = _sc2_call(src, dst, sd2[:, 0], sd2[:, 1])

    out = _tc3_call(h2, w.reshape(NP, 1), b2.reshape(1, O))
    return out.reshape(O)
